# Initial kernel scaffold; baseline (speedup 1.0000x reference)
import jax
import jax.numpy as jnp
from jax.experimental import pallas as pl


def kernel(W_in, indices):
    raise NotImplementedError("write your pallas kernel here")


# speedup vs baseline: 1.6823x; 1.6823x over previous
import functools

import jax
import jax.numpy as jnp
from jax import lax
from jax.experimental import pallas as pl
from jax.experimental.pallas import tpu as pltpu
from jax.experimental.pallas import tpu_sc as plsc

_NC = 2
_NS = 16
_NW = _NC * _NS


def kernel(W_in, indices):
    V, D = W_in.shape
    B, L = indices.shape
    N = B * L
    C = 128
    per_w = N // _NW
    nch = per_w // C
    assert per_w * _NW == N and nch * C == per_w

    idx3 = indices.reshape(_NW, nch, C)

    mesh = plsc.VectorSubcoreMesh(core_axis_name="c", subcore_axis_name="s")

    @functools.partial(
        pl.kernel,
        mesh=mesh,
        out_type=jax.ShapeDtypeStruct((_NW, nch, C, D), jnp.float32),
        scratch_types=[
            pltpu.VMEM((nch, C), jnp.int32),
            pltpu.VMEM((C, D), jnp.float32),
            pltpu.SemaphoreType.DMA,
        ],
        compiler_params=pltpu.CompilerParams(use_tc_tiling_on_sc=False),
    )
    def gather_kernel(table, idx, out, idx_v, rows_v, sem):
        wid = lax.axis_index("s") * _NC + lax.axis_index("c")
        pltpu.sync_copy(idx.at[wid], idx_v)

        def body(j, carry):
            pltpu.async_copy(table.at[idx_v.at[j]], rows_v, sem).wait()
            pltpu.sync_copy(rows_v, out.at[wid, j])
            return carry

        lax.fori_loop(0, nch, body, 0)

    out = gather_kernel(W_in, idx3)
    return out.reshape(B, L, D)


# speedup vs baseline: 1.8763x; 1.1153x over previous
import functools

import jax
import jax.numpy as jnp
from jax import lax
from jax.experimental import pallas as pl
from jax.experimental.pallas import tpu as pltpu
from jax.experimental.pallas import tpu_sc as plsc

_NC = 2
_NS = 16
_NW = _NC * _NS


def kernel(W_in, indices):
    V, D = W_in.shape
    B, L = indices.shape
    N = B * L
    C = 128
    per_w = N // _NW
    nch = per_w // C
    assert per_w * _NW == N and nch * C == per_w

    idx3 = indices.reshape(_NW, nch, C)

    mesh = plsc.VectorSubcoreMesh(core_axis_name="c", subcore_axis_name="s")

    NBUF = 8
    ngrp = nch // NBUF
    assert ngrp * NBUF == nch

    @functools.partial(
        pl.kernel,
        mesh=mesh,
        out_type=jax.ShapeDtypeStruct((_NW, nch, C, D), jnp.float32),
        scratch_types=[
            pltpu.VMEM((nch, C), jnp.int32),
            pltpu.VMEM((NBUF, C, D), jnp.float32),
            [pltpu.SemaphoreType.DMA] * NBUF,
            [pltpu.SemaphoreType.DMA] * NBUF,
        ],
        compiler_params=pltpu.CompilerParams(use_tc_tiling_on_sc=False),
    )
    def gather_kernel(table, idx, out, idx_v, rows_v, gsem, ssem):
        wid = lax.axis_index("s") * _NC + lax.axis_index("c")
        pltpu.sync_copy(idx.at[wid], idx_v)

        for b in range(NBUF):
            pltpu.async_copy(table.at[idx_v.at[b]], rows_v.at[b], gsem[b])

        def body(g, carry):
            for b in range(NBUF):
                j = g * NBUF + b
                pltpu.make_async_copy(
                    table.at[idx_v.at[b]], rows_v.at[b], gsem[b]
                ).wait()
                pltpu.async_copy(rows_v.at[b], out.at[wid, j], ssem[b])

                @pl.when(g < ngrp - 1)
                def _():
                    pltpu.make_async_copy(
                        rows_v.at[b], out.at[wid, j], ssem[b]
                    ).wait()
                    pltpu.async_copy(
                        table.at[idx_v.at[j + NBUF]], rows_v.at[b], gsem[b]
                    )

            return carry

        lax.fori_loop(0, ngrp, body, 0)

        for b in range(NBUF):
            pltpu.make_async_copy(
                rows_v.at[b], out.at[wid, nch - NBUF + b], ssem[b]
            ).wait()

    out = gather_kernel(W_in, idx3)
    return out.reshape(B, L, D)
